# baseline (speedup 1.0000x reference)
import functools

import jax
import jax.numpy as jnp
from jax import lax
from jax.experimental import pallas as pl
from jax.experimental.pallas import tpu as pltpu
from jax.experimental.pallas import tpu_sc as plsc

N = 10000
T = 12
C_IN = 128
HID = 256
C_OUT = 128
K = 3
TP = T - K + 1

NC = 2
NS = 16
L = 16
NW = NC * NS
CHUNK = 128
DW = 128
NPAD = 10112
STRIPE = NPAD // NS

NB = 1000
GRID = N // NB

def _sc_mesh():
    return plsc.VectorSubcoreMesh(
        core_axis_name="c", subcore_axis_name="s", num_cores=NC, num_subcores=NS)


def _zero_fill(buf, width):
    zero = jnp.zeros((L,), jnp.float32)

    def fill_zero(i, _):
        for k in range(width // L):
            buf[i, L * k:L * (k + 1)] = zero
        return 0

    lax.fori_loop(0, CHUNK, fill_zero, 0)


def _zero_stripe(buf, shared, s):
    base = s * STRIPE
    for z in range(STRIPE // CHUNK):
        pltpu.sync_copy(buf, shared.at[pl.ds(base + z * CHUNK, CHUNK)])
    rem = STRIPE % CHUNK
    if rem:
        pltpu.sync_copy(buf.at[pl.ds(0, rem)],
                        shared.at[pl.ds(base + (STRIPE // CHUNK) * CHUNK, rem)])


def _deg_body(nchunks, dst_hbm, out_hbm,
              idx0, idx1, idx2, idx3, sem0, sem1, sem2, sem3, ones_v, deg_sp):
    c = lax.axis_index("c")
    s = lax.axis_index("s")
    wid = c * NS + s
    one = jnp.full((L,), 1.0, jnp.float32)

    _zero_fill(ones_v, DW)
    _zero_stripe(ones_v, deg_sp, s)

    def fill_ones(i, _):
        for k in range(DW // L):
            ones_v[i, L * k:L * (k + 1)] = one
        return 0

    lax.fori_loop(0, CHUNK, fill_ones, 0)
    plsc.subcore_barrier()

    slots = ((idx0, sem0), (idx1, sem1), (idx2, sem2), (idx3, sem3))
    nslots = len(slots)

    def body(jj, _):
        for ph, (idx_v, sem) in enumerate(slots):
            j = nslots * jj + ph

            @pl.when(j >= nslots)
            def _():
                pltpu.make_async_copy(ones_v, deg_sp.at[idx_v], sem).wait()

            pltpu.sync_copy(
                dst_hbm.at[pl.ds((wid * nchunks + j) * CHUNK, CHUNK)], idx_v)
            pltpu.async_copy(ones_v, deg_sp.at[idx_v], sem, add=True)
        return 0

    lax.fori_loop(0, nchunks // nslots, body, 0)
    for idx_v, sem in slots:
        pltpu.make_async_copy(ones_v, deg_sp.at[idx_v], sem).wait()
    plsc.subcore_barrier()
    pltpu.sync_copy(deg_sp.at[pl.ds(s * STRIPE, STRIPE)],
                    out_hbm.at[c, pl.ds(s * STRIPE, STRIPE)])


def _scatter_body(nchunks, m_hbm, src_hbm, dst_hbm, out_hbm,
                  srcx_a, dstx_a, srcx_b, dstx_b, rows_a, rows_b,
                  semg_a, semg_b, acc_sp):
    c = lax.axis_index("c")
    s = lax.axis_index("s")
    wid = c * NS + s

    _zero_fill(rows_a, C_OUT)
    _zero_stripe(rows_a, acc_sp, s)
    plsc.subcore_barrier()

    slots = ((srcx_a, dstx_a, rows_a, semg_a),
             (srcx_b, dstx_b, rows_b, semg_b))

    base0 = wid * nchunks * CHUNK
    pltpu.sync_copy(src_hbm.at[pl.ds(base0, CHUNK)], srcx_a)
    pltpu.sync_copy(dst_hbm.at[pl.ds(base0, CHUNK)], dstx_a)
    pltpu.async_copy(m_hbm.at[srcx_a], rows_a, semg_a)

    def phase(j, mine, other):
        m_src, m_dst, m_rows, m_semg = mine
        o_src, o_dst, o_rows, o_semg = other

        @pl.when(j + 1 < nchunks)
        def _():
            base = (wid * nchunks + j + 1) * CHUNK
            pltpu.sync_copy(src_hbm.at[pl.ds(base, CHUNK)], o_src)
            pltpu.sync_copy(dst_hbm.at[pl.ds(base, CHUNK)], o_dst)
            pltpu.async_copy(m_hbm.at[o_src], o_rows, o_semg)

        pltpu.make_async_copy(m_hbm.at[m_src], m_rows, m_semg).wait()
        pltpu.sync_copy(m_rows, acc_sp.at[m_dst], add=True)

    def body(jj, _):
        phase(2 * jj, slots[0], slots[1])
        phase(2 * jj + 1, slots[1], slots[0])
        return 0

    lax.fori_loop(0, nchunks // 2, body, 0)
    plsc.subcore_barrier()
    pltpu.sync_copy(acc_sp.at[pl.ds(s * STRIPE, STRIPE)],
                    out_hbm.at[c, pl.ds(s * STRIPE, STRIPE)])


def _dense_body(x_ref, w1_ref, w2_ref, w3_ref, b_ref, gw_ref, degp_ref, m_ref):
    xb = x_ref[...]
    b1 = b_ref[0:1, :]
    b2 = b_ref[1:2, :]
    b3 = b_ref[2:3, :]
    w1 = w1_ref[...]
    w2 = w2_ref[...]
    w3 = w3_ref[...]
    y = [xb[:, t, :] for t in range(T)]
    acc = jnp.zeros((NB, HID), jnp.float32)
    for t in range(TP):
        xw = jnp.concatenate([y[t], y[t + 1], y[t + 2]], axis=1)
        p = jnp.dot(xw, w1, preferred_element_type=jnp.float32) + b1
        q = jnp.dot(xw, w2, preferred_element_type=jnp.float32) + b2
        r = jnp.dot(xw, w3, preferred_element_type=jnp.float32) + b3
        acc = acc + jnp.maximum(p * jax.nn.sigmoid(q) + r, 0.0)
    h = acc * jnp.float32(1.0 / TP)
    h2 = jnp.dot(h, gw_ref[...], preferred_element_type=jnp.float32)
    deg = degp_ref[0, :, 0:1] + degp_ref[1, :, 0:1] + 1.0
    m_ref[...] = h2 * lax.rsqrt(deg)


def _final_body(accp_ref, m_ref, degp_ref, gb_ref, out_ref):
    deg = degp_ref[0, :, 0:1] + degp_ref[1, :, 0:1] + 1.0
    dinv = lax.rsqrt(deg)
    tot = accp_ref[0] + accp_ref[1] + m_ref[...]
    out_ref[...] = jnp.maximum(tot * dinv + gb_ref[...], 0.0)


def _make_deg_call(nchunks):
    return pl.kernel(
        functools.partial(_deg_body, nchunks),
        out_type=jax.ShapeDtypeStruct((NC, NPAD, DW), jnp.float32),
        mesh=_sc_mesh(),
        scratch_types=[
            pltpu.VMEM((CHUNK,), jnp.int32),
            pltpu.VMEM((CHUNK,), jnp.int32),
            pltpu.VMEM((CHUNK,), jnp.int32),
            pltpu.VMEM((CHUNK,), jnp.int32),
            pltpu.SemaphoreType.DMA,
            pltpu.SemaphoreType.DMA,
            pltpu.SemaphoreType.DMA,
            pltpu.SemaphoreType.DMA,
            pltpu.VMEM((CHUNK, DW), jnp.float32),
            pltpu.VMEM_SHARED((NPAD, DW), jnp.float32),
        ],
    )


def _make_scatter_call(nchunks):
    return pl.kernel(
        functools.partial(_scatter_body, nchunks),
        out_type=jax.ShapeDtypeStruct((NC, NPAD, C_OUT), jnp.float32),
        mesh=_sc_mesh(),
        scratch_types=[
            pltpu.VMEM((CHUNK,), jnp.int32),
            pltpu.VMEM((CHUNK,), jnp.int32),
            pltpu.VMEM((CHUNK,), jnp.int32),
            pltpu.VMEM((CHUNK,), jnp.int32),
            pltpu.VMEM((CHUNK, C_OUT), jnp.float32),
            pltpu.VMEM((CHUNK, C_OUT), jnp.float32),
            pltpu.SemaphoreType.DMA,
            pltpu.SemaphoreType.DMA,
            pltpu.VMEM_SHARED((NPAD, C_OUT), jnp.float32),
        ],
    )


def _dense_call(x, W1, W2, W3, bstack, gw, degp):
    return pl.pallas_call(
        _dense_body,
        grid=(GRID,),
        in_specs=[
            pl.BlockSpec((NB, T, C_IN), lambda i: (i, 0, 0)),
            pl.BlockSpec((K * C_IN, HID), lambda i: (0, 0)),
            pl.BlockSpec((K * C_IN, HID), lambda i: (0, 0)),
            pl.BlockSpec((K * C_IN, HID), lambda i: (0, 0)),
            pl.BlockSpec((4, HID), lambda i: (0, 0)),
            pl.BlockSpec((HID, C_OUT), lambda i: (0, 0)),
            pl.BlockSpec((NC, NB, DW), lambda i: (0, i, 0)),
        ],
        out_specs=pl.BlockSpec((NB, C_OUT), lambda i: (i, 0)),
        out_shape=jax.ShapeDtypeStruct((N, C_OUT), jnp.float32),
    )(x, W1, W2, W3, bstack, gw, degp)


def _final_call(accp, m, degp, gb2):
    return pl.pallas_call(
        _final_body,
        grid=(GRID,),
        in_specs=[
            pl.BlockSpec((NC, NB, C_OUT), lambda i: (0, i, 0)),
            pl.BlockSpec((NB, C_OUT), lambda i: (i, 0)),
            pl.BlockSpec((NC, NB, DW), lambda i: (0, i, 0)),
            pl.BlockSpec((1, C_OUT), lambda i: (0, 0)),
        ],
        out_specs=pl.BlockSpec((NB, C_OUT), lambda i: (i, 0)),
        out_shape=jax.ShapeDtypeStruct((N, C_OUT), jnp.float32),
    )(accp, m, degp, gb2)


def kernel(x, edge_index, w1, b1, w2, b2, w3, b3, gw, gb):
    E = edge_index.shape[1]
    ept = -(-E // NW)
    nchunks = -(-ept // CHUNK)
    nchunks = -(-nchunks // 4) * 4
    ept_pad = nchunks * CHUNK
    pad = NW * ept_pad - E

    trash = N + (jnp.arange(pad, dtype=jnp.int32) % (NPAD - N))
    src = jnp.concatenate([edge_index[0], jnp.zeros((pad,), jnp.int32)])
    dst = jnp.concatenate([edge_index[1], trash])


    W1 = jnp.transpose(w1[:, :, 0, :], (2, 1, 0)).reshape(K * C_IN, HID)
    W2 = jnp.transpose(w2[:, :, 0, :], (2, 1, 0)).reshape(K * C_IN, HID)
    W3 = jnp.transpose(w3[:, :, 0, :], (2, 1, 0)).reshape(K * C_IN, HID)
    bstack = jnp.stack([b1, b2, b3, jnp.zeros_like(b1)])

    degp = _make_deg_call(nchunks)(dst)
    degp_n = degp[:, :N, :]

    m = _dense_call(x, W1, W2, W3, bstack, gw, degp_n)

    accp = _make_scatter_call(nchunks)(m, src, dst)

    out = _final_call(accp[:, :N, :], m, degp_n, gb[None, :])
    return out[None]
